# Initial kernel scaffold; baseline (speedup 1.0000x reference)
import jax
import jax.numpy as jnp
from jax.experimental import pallas as pl


def kernel(x, edge_index, edge_attr, u, batch, We, ge, be, ae, Wn1, gn1, bn1, an1, Wn2, gn2, bn2, an2, Wg, gg, bg, ag):
    raise NotImplementedError("write your pallas kernel here")


# speedup vs baseline: 1.0376x; 1.0376x over previous
import functools

import jax
import jax.numpy as jnp
from jax import lax
from jax.experimental import pallas as pl
from jax.experimental.pallas import tpu as pltpu

_N = 10000
_E = 320000
_NG = 8
_NF = 128
_EF = 16
_GF = 16

_EB = 1000
_EF8 = _E // 8
_GRID = _EF8 // _EB


def _prelu(z, a):
    return jnp.maximum(z, 0.0) + a * jnp.minimum(z, 0.0)


def _k0a_body(x_ref, u_ref, batch_ref, wex_ref, weu_ref, wa_ref,
              t1_ref, ta_ref, xa_ref):
    x = x_ref[...]
    t1 = 0.5 * jnp.dot(x, wex_ref[...].T, preferred_element_type=jnp.float32)
    oh = (batch_ref[...] == lax.broadcasted_iota(jnp.int32, (1, _NG), 1)
          ).astype(jnp.float32)
    ub = jnp.dot(oh, u_ref[...], preferred_element_type=jnp.float32)
    ta = t1 + jnp.dot(ub, weu_ref[...].T, preferred_element_type=jnp.float32)
    xa = jnp.dot(x, wa_ref[...].T, preferred_element_type=jnp.float32)
    t1_ref[...] = t1
    ta_ref[...] = ta
    xa_ref[...] = xa


def _k0a(x, u, batch2d, We_x, We_u, A):
    return pl.pallas_call(
        _k0a_body,
        out_shape=(
            jax.ShapeDtypeStruct((_N, _EF), jnp.float32),
            jax.ShapeDtypeStruct((_N, _EF), jnp.float32),
            jax.ShapeDtypeStruct((_N, _NF), jnp.float32),
        ),
    )(x, u, batch2d, We_x, We_u, A)


def _k0b_body(eattr_ref, m0_ref, out_ref):
    out_ref[...] = jnp.dot(eattr_ref[...], m0_ref[...],
                           preferred_element_type=jnp.float32)


def _k0b(eattr_f, M0):
    return pl.pallas_call(
        _k0b_body,
        grid=(_GRID,),
        in_specs=[
            pl.BlockSpec((_EB, 128), lambda i: (i, 0)),
            pl.BlockSpec((128, 128), lambda i: (0, 0)),
        ],
        out_specs=pl.BlockSpec((_EB, 128), lambda i: (i, 0)),
        out_shape=jax.ShapeDtypeStruct((_EF8, 128), jnp.float32),
    )(eattr_f, M0)


def _k2_body(p_ref, s1_ref, t1_ref, a_ref, e2_ref, r_ref, se_ref):
    i = pl.program_id(0)
    y = p_ref[...] * s1_ref[...] + t1_ref[...]
    a = a_ref[0, 0]
    e2 = jnp.maximum(y, 0.0) + a * jnp.minimum(y, 0.0)
    e2_ref[...] = e2

    @pl.when(i == 0)
    def _():
        r_ref[...] = jnp.zeros_like(r_ref)
        se_ref[...] = jnp.zeros_like(se_ref)

    r_ref[...] += lax.dot_general(e2, e2, (((0,), (0,)), ((), ())),
                                  preferred_element_type=jnp.float32)
    se_ref[...] += jnp.sum(e2, axis=0, keepdims=True)


def _k2(p_f, s1_tiled, t1_tiled, ae):
    return pl.pallas_call(
        _k2_body,
        grid=(_GRID,),
        in_specs=[
            pl.BlockSpec((_EB, 128), lambda i: (i, 0)),
            pl.BlockSpec((1, 128), lambda i: (0, 0)),
            pl.BlockSpec((1, 128), lambda i: (0, 0)),
            pl.BlockSpec((1, 1), lambda i: (0, 0)),
        ],
        out_specs=(
            pl.BlockSpec((_EB, 128), lambda i: (i, 0)),
            pl.BlockSpec((128, 128), lambda i: (0, 0)),
            pl.BlockSpec((1, 128), lambda i: (0, 0)),
        ),
        out_shape=(
            jax.ShapeDtypeStruct((_EF8, 128), jnp.float32),
            jax.ShapeDtypeStruct((128, 128), jnp.float32),
            jax.ShapeDtypeStruct((1, 128), jnp.float32),
        ),
    )(p_f, s1_tiled, t1_tiled, ae)


def _k4_body(cntp_ref, xa_ref, sc2p_ref, s2_ref, se2_ref, bt_ref,
             gn1_ref, bn1_ref,
             xas_ref, s3_ref, tt3_ref, cnt2_ref):
    cnt2 = jnp.sum(cntp_ref[...], axis=0)
    cnt2_ref[...] = cnt2
    cc = cnt2[1:2, :]
    xa = xa_ref[...]
    sc2 = sc2p_ref[0] + sc2p_ref[1]
    bt = bt_ref[...]
    sum_h = (jnp.dot(cc, xa, preferred_element_type=jnp.float32)
             + jnp.dot(se2_ref[...], bt, preferred_element_type=jnp.float32))
    sq1 = jnp.dot(cc, xa * xa, preferred_element_type=jnp.float32)
    ct = lax.dot_general(sc2, xa, (((0,), (0,)), ((), ())),
                         preferred_element_type=jnp.float32)
    sq2 = 2.0 * jnp.sum(bt * ct, axis=0, keepdims=True)
    q = jnp.dot(s2_ref[...], bt, preferred_element_type=jnp.float32)
    sq3 = jnp.sum(bt * q, axis=0, keepdims=True)
    inv_e = 1.0 / _E
    mean3 = sum_h * inv_e
    var3 = (sq1 + sq2 + sq3) * inv_e - mean3 * mean3
    s3 = gn1_ref[...] * lax.rsqrt(var3 + 1e-5)
    tt3 = bn1_ref[...] - mean3 * s3
    s3_ref[...] = s3
    tt3_ref[...] = tt3
    xas_ref[...] = xa * s3


def _k4(cntP, xa, sc2P, S2, sum_e2, Bt, gn1, bn1):
    return pl.pallas_call(
        _k4_body,
        out_shape=(
            jax.ShapeDtypeStruct((_N, _NF), jnp.float32),
            jax.ShapeDtypeStruct((1, _NF), jnp.float32),
            jax.ShapeDtypeStruct((1, _NF), jnp.float32),
            jax.ShapeDtypeStruct((2, _N), jnp.float32),
        ),
    )(cntP, xa, sc2P, S2, sum_e2, Bt, gn1, bn1)


def _k5_body(e2_ref, m5_ref, tt3_ref, out_ref):
    out_ref[...] = (jnp.dot(e2_ref[...], m5_ref[...],
                            preferred_element_type=jnp.float32)
                    + tt3_ref[...])


def _k5(e2_f, M5, tt3_tiled):
    return pl.pallas_call(
        _k5_body,
        grid=(_GRID,),
        in_specs=[
            pl.BlockSpec((_EB, 128), lambda i: (i, 0)),
            pl.BlockSpec((128, 1024), lambda i: (0, 0)),
            pl.BlockSpec((1, 1024), lambda i: (0, 0)),
        ],
        out_specs=pl.BlockSpec((_EB, 1024), lambda i: (i, 0)),
        out_shape=jax.ShapeDtypeStruct((_EF8, 1024), jnp.float32),
    )(e2_f, M5, tt3_tiled)


def _k7_body(shp_ref, cnt_ref, u_ref, batch_ref, wn2x_ref, wn2u_ref,
             gn2_ref, bn2_ref, an2_ref, wgu_ref, wgx_ref,
             gg_ref, bg_ref, ag_ref,
             x2_ref, u2_ref):
    sh = shp_ref[0] + shp_ref[1]
    hm = sh / jnp.maximum(cnt_ref[...], 1.0)
    oh = (batch_ref[...] == lax.broadcasted_iota(jnp.int32, (1, _NG), 1)
          ).astype(jnp.float32)
    ub = jnp.dot(oh, u_ref[...], preferred_element_type=jnp.float32)
    y = (jnp.dot(hm, wn2x_ref[...].T, preferred_element_type=jnp.float32)
         + jnp.dot(ub, wn2u_ref[...].T, preferred_element_type=jnp.float32))
    m = jnp.mean(y, axis=0, keepdims=True)
    yc = y - m
    v = jnp.mean(yc * yc, axis=0, keepdims=True)
    y = yc * lax.rsqrt(v + 1e-5) * gn2_ref[...] + bn2_ref[...]
    a2 = an2_ref[0, 0]
    x2 = jnp.maximum(y, 0.0) + a2 * jnp.minimum(y, 0.0)
    x2_ref[...] = x2

    segn = lax.dot_general(oh, x2, (((0,), (0,)), ((), ())),
                           preferred_element_type=jnp.float32)
    cntb = jnp.sum(oh, axis=0)[:, None]
    segm = segn / jnp.maximum(cntb, 1.0)
    yg = (jnp.dot(u_ref[...], wgu_ref[...].T, preferred_element_type=jnp.float32)
          + jnp.dot(segm, wgx_ref[...].T, preferred_element_type=jnp.float32))
    mg = jnp.mean(yg, axis=0, keepdims=True)
    ygc = yg - mg
    vg = jnp.mean(ygc * ygc, axis=0, keepdims=True)
    yg = ygc * lax.rsqrt(vg + 1e-5) * gg_ref[...] + bg_ref[...]
    ag = ag_ref[0, 0]
    u2_ref[...] = jnp.maximum(yg, 0.0) + ag * jnp.minimum(yg, 0.0)


def _k7(shP, cnt_row, u, batch2d, Wn2x, Wn2u, gn2, bn2, an2,
        Wgu, Wgx, gg, bg, ag):
    return pl.pallas_call(
        _k7_body,
        out_shape=(
            jax.ShapeDtypeStruct((_N, _NF), jnp.float32),
            jax.ShapeDtypeStruct((_NG, _GF), jnp.float32),
        ),
    )(shP, cnt_row, u, batch2d, Wn2x, Wn2u, gn2, bn2, an2,
      Wgu, Wgx, gg, bg, ag)


def kernel(x, edge_index, edge_attr, u, batch,
           We, ge, be, ae,
           Wn1, gn1, bn1, an1,
           Wn2, gn2, bn2, an2,
           Wg, gg, bg, ag):
    row = edge_index[0]
    col = edge_index[1]
    batch2d = batch[:, None].astype(jnp.int32)

    We_x = We[:, :_NF]
    We_e = We[:, _NF:_NF + _EF]
    We_u = We[:, _NF + _EF:]
    A = Wn1[:, :_NF]
    Bt = Wn1[:, _NF:].T
    eye8 = jnp.eye(8, dtype=jnp.float32)
    M0 = jnp.einsum('ij,fg->ifjg', eye8, We_e.T).reshape(128, 128)
    Wn2x = Wn2[:, :_NF]
    Wn2u = Wn2[:, _NF:]
    Wgu = Wg[:, :_GF]
    Wgx = Wg[:, _GF:]
    r2 = lambda v: v[None, :]
    r11 = lambda s: jnp.asarray(s, jnp.float32).reshape(1, 1)

    t1, ta, xa = _k0a(x, u, batch2d, We_x, We_u, A)
    ea_f = _k0b(edge_attr.reshape(_EF8, 128), M0)
    ea = ea_f.reshape(_E, _EF)

    p = ta[row] + t1[col] + ea
    sp = jnp.sum(p, axis=0)
    spp = jnp.sum(p * p, axis=0)
    ones_e = jnp.ones((_E,), jnp.float32)
    cnt_row = jax.ops.segment_sum(ones_e, row, num_segments=_N)
    cnt_col = jax.ops.segment_sum(ones_e, col, num_segments=_N)
    cntP = jnp.zeros((32, 2, _N), jnp.float32)
    cntP = cntP.at[0, 0].set(cnt_row).at[0, 1].set(cnt_col)

    mean1 = sp / _E
    var1 = spp / _E - mean1 * mean1
    s1 = ge * lax.rsqrt(var1 + 1e-5)
    t1bn = be - mean1 * s1

    e2_f, R, se_row = _k2(p.reshape(_EF8, 128),
                          jnp.tile(s1, 8)[None, :],
                          jnp.tile(t1bn, 8)[None, :], r11(ae))
    e2 = e2_f.reshape(_E, _EF)
    S2 = jnp.einsum('ifig->fg', R.reshape(8, 16, 8, 16))
    sum_e2 = jnp.sum(se_row.reshape(8, 16), axis=0, keepdims=True)

    sc2 = jax.ops.segment_sum(e2, col, num_segments=_N)
    sc2P = jnp.zeros((2, _N, _EF), jnp.float32).at[0].set(sc2)

    xas, s3, tt3, cnt2 = _k4(cntP, xa, sc2P, S2, sum_e2, Bt, r2(gn1), r2(bn1))

    Bs_t = Bt * s3.reshape(-1)[None, :]
    M5 = jnp.einsum('ji,of->jfio', eye8,
                    Bs_t.T).reshape(128, 1024)
    w_f = _k5(e2_f, M5, jnp.tile(tt3.reshape(-1), 8)[None, :])
    w = w_f.reshape(_E, _NF)

    z = xas[col] + w
    c1 = (1.0 + an1) * 0.5
    c2 = (1.0 - an1) * 0.5
    h = c1 * z + c2 * jnp.abs(z)
    sh = jax.ops.segment_sum(h, row, num_segments=_N)
    shP = jnp.zeros((2, _N, _NF), jnp.float32).at[0].set(sh)

    x2, u2 = _k7(shP, cnt2[0][:, None], u, batch2d, Wn2x, Wn2u,
                 r2(gn2), r2(bn2), r11(an2), Wgu, Wgx,
                 r2(gg), r2(bg), r11(ag))
    return (x2, e2, u2)


# speedup vs baseline: 1.2813x; 1.2348x over previous
import functools

import jax
import jax.numpy as jnp
from jax import lax
from jax.experimental import pallas as pl
from jax.experimental.pallas import tpu as pltpu
from jax.experimental.pallas import tpu_sc as plsc

_N = 10000
_E = 320000
_NG = 8
_NF = 128
_EF = 16
_GF = 16

_EB = 1000
_EF8 = _E // 8
_GRID = _EF8 // _EB

_NW = 32
_EW = _E // _NW
_CH = 80
_NCH = _EW // _CH
_NSL = _N // 16
_SC_MESH = plsc.VectorSubcoreMesh(core_axis_name="c", subcore_axis_name="s")


def _prelu(z, a):
    return jnp.maximum(z, 0.0) + a * jnp.minimum(z, 0.0)


def _k0a_body(x_ref, u_ref, batch_ref, wex_ref, weu_ref, wa_ref,
              t1_ref, ta_ref, xa_ref):
    x = x_ref[...]
    t1 = 0.5 * jnp.dot(x, wex_ref[...].T, preferred_element_type=jnp.float32)
    oh = (batch_ref[...] == lax.broadcasted_iota(jnp.int32, (1, _NG), 1)
          ).astype(jnp.float32)
    ub = jnp.dot(oh, u_ref[...], preferred_element_type=jnp.float32)
    ta = t1 + jnp.dot(ub, weu_ref[...].T, preferred_element_type=jnp.float32)
    xa = jnp.dot(x, wa_ref[...].T, preferred_element_type=jnp.float32)
    t1_ref[...] = t1
    ta_ref[...] = ta
    xa_ref[...] = xa


def _k0a(x, u, batch2d, We_x, We_u, A):
    return pl.pallas_call(
        _k0a_body,
        out_shape=(
            jax.ShapeDtypeStruct((_N, _EF), jnp.float32),
            jax.ShapeDtypeStruct((_N, _EF), jnp.float32),
            jax.ShapeDtypeStruct((_N, _NF), jnp.float32),
        ),
    )(x, u, batch2d, We_x, We_u, A)


def _k0b_body(eattr_ref, m0_ref, out_ref):
    out_ref[...] = jnp.dot(eattr_ref[...], m0_ref[...],
                           preferred_element_type=jnp.float32)


def _k0b(eattr_f, M0):
    return pl.pallas_call(
        _k0b_body,
        grid=(_GRID,),
        in_specs=[
            pl.BlockSpec((_EB, 128), lambda i: (i, 0)),
            pl.BlockSpec((128, 128), lambda i: (0, 0)),
        ],
        out_specs=pl.BlockSpec((_EB, 128), lambda i: (i, 0)),
        out_shape=jax.ShapeDtypeStruct((_EF8, 128), jnp.float32),
    )(eattr_f, M0)


def _k2_body(p_ref, s1_ref, t1_ref, a_ref, e2_ref, r_ref, se_ref):
    i = pl.program_id(0)
    y = p_ref[...] * s1_ref[...] + t1_ref[...]
    a = a_ref[0, 0]
    e2 = jnp.maximum(y, 0.0) + a * jnp.minimum(y, 0.0)
    e2_ref[...] = e2

    @pl.when(i == 0)
    def _():
        r_ref[...] = jnp.zeros_like(r_ref)
        se_ref[...] = jnp.zeros_like(se_ref)

    r_ref[...] += lax.dot_general(e2, e2, (((0,), (0,)), ((), ())),
                                  preferred_element_type=jnp.float32)
    se_ref[...] += jnp.sum(e2, axis=0, keepdims=True)


def _k2(p_f, s1_tiled, t1_tiled, ae):
    return pl.pallas_call(
        _k2_body,
        grid=(_GRID,),
        in_specs=[
            pl.BlockSpec((_EB, 128), lambda i: (i, 0)),
            pl.BlockSpec((1, 128), lambda i: (0, 0)),
            pl.BlockSpec((1, 128), lambda i: (0, 0)),
            pl.BlockSpec((1, 1), lambda i: (0, 0)),
        ],
        out_specs=(
            pl.BlockSpec((_EB, 128), lambda i: (i, 0)),
            pl.BlockSpec((128, 128), lambda i: (0, 0)),
            pl.BlockSpec((1, 128), lambda i: (0, 0)),
        ),
        out_shape=(
            jax.ShapeDtypeStruct((_EF8, 128), jnp.float32),
            jax.ShapeDtypeStruct((128, 128), jnp.float32),
            jax.ShapeDtypeStruct((1, 128), jnp.float32),
        ),
    )(p_f, s1_tiled, t1_tiled, ae)


def _k4_body(cntp_ref, xa_ref, sc2p_ref, s2_ref, se2_ref, bt_ref,
             gn1_ref, bn1_ref,
             xas_ref, s3_ref, tt3_ref, cnt2_ref):
    cnt2 = jnp.sum(cntp_ref[...], axis=0)
    cnt2_ref[...] = cnt2
    cc = cnt2[1:2, :]
    xa = xa_ref[...]
    sc2 = sc2p_ref[0] + sc2p_ref[1]
    bt = bt_ref[...]
    sum_h = (jnp.dot(cc, xa, preferred_element_type=jnp.float32)
             + jnp.dot(se2_ref[...], bt, preferred_element_type=jnp.float32))
    sq1 = jnp.dot(cc, xa * xa, preferred_element_type=jnp.float32)
    ct = lax.dot_general(sc2, xa, (((0,), (0,)), ((), ())),
                         preferred_element_type=jnp.float32)
    sq2 = 2.0 * jnp.sum(bt * ct, axis=0, keepdims=True)
    q = jnp.dot(s2_ref[...], bt, preferred_element_type=jnp.float32)
    sq3 = jnp.sum(bt * q, axis=0, keepdims=True)
    inv_e = 1.0 / _E
    mean3 = sum_h * inv_e
    var3 = (sq1 + sq2 + sq3) * inv_e - mean3 * mean3
    s3 = gn1_ref[...] * lax.rsqrt(var3 + 1e-5)
    tt3 = bn1_ref[...] - mean3 * s3
    s3_ref[...] = s3
    tt3_ref[...] = tt3
    xas_ref[...] = xa * s3


def _k4(cntP, xa, sc2P, S2, sum_e2, Bt, gn1, bn1):
    return pl.pallas_call(
        _k4_body,
        out_shape=(
            jax.ShapeDtypeStruct((_N, _NF), jnp.float32),
            jax.ShapeDtypeStruct((1, _NF), jnp.float32),
            jax.ShapeDtypeStruct((1, _NF), jnp.float32),
            jax.ShapeDtypeStruct((2, _N), jnp.float32),
        ),
    )(cntP, xa, sc2P, S2, sum_e2, Bt, gn1, bn1)


def _zero_acc(zero_v, acc_sh, sid, width):
    def zrow(i, _):
        zero_v[i, :] = jnp.zeros((width,), jnp.float32)
        return 0
    lax.fori_loop(0, 48, zrow, 0)

    def zchunk(k, _):
        pltpu.sync_copy(zero_v, acc_sh.at[pl.ds(sid * 624 + k * 48, 48)])
        return 0
    lax.fori_loop(0, 13, zchunk, 0)

    @pl.when(sid == 15)
    def _():
        pltpu.sync_copy(zero_v.at[pl.ds(0, 16)], acc_sh.at[pl.ds(9984, 16)])


def _read_acc(acc_sh, out_hbm, cid, sid):
    def rchunk(k, _):
        off = sid * 624 + k * 48
        pltpu.sync_copy(acc_sh.at[pl.ds(off, 48)],
                        out_hbm.at[cid, pl.ds(off, 48)])
        return 0
    lax.fori_loop(0, 13, rchunk, 0)

    @pl.when(sid == 15)
    def _():
        pltpu.sync_copy(acc_sh.at[pl.ds(9984, 16)],
                        out_hbm.at[cid, pl.ds(9984, 16)])


def _k3_body(e2_hbm, col_hbm, out_hbm, idx_v, e2_v, zero_v, acc_sh, sem):
    cid = lax.axis_index("c")
    sid = lax.axis_index("s")
    wid = cid * 16 + sid

    _zero_acc(zero_v, acc_sh, sid, _EF)
    plsc.subcore_barrier()

    def chunk(j, _):
        base = wid * _EW + j * _CH
        pltpu.sync_copy(col_hbm.at[pl.ds(base, _CH)], idx_v)
        pltpu.sync_copy(e2_hbm.at[pl.ds(base, _CH)], e2_v)
        pltpu.sync_copy(e2_v, acc_sh.at[idx_v], add=True)
        return 0
    lax.fori_loop(0, _NCH, chunk, 0)
    plsc.subcore_barrier()

    _read_acc(acc_sh, out_hbm, cid, sid)


_k3 = functools.partial(
    pl.kernel, _k3_body, mesh=_SC_MESH,
    out_type=jax.ShapeDtypeStruct((2, _N, _EF), jnp.float32),
    scratch_types=[
        pltpu.VMEM((_CH,), jnp.int32),
        pltpu.VMEM((_CH, _EF), jnp.float32),
        pltpu.VMEM((48, _EF), jnp.float32),
        pltpu.VMEM_SHARED((_N, _EF), jnp.float32),
        pltpu.SemaphoreType.DMA,
    ],
)()


def _k5_body(e2_ref, m5_ref, tt3_ref, out_ref):
    out_ref[...] = (jnp.dot(e2_ref[...], m5_ref[...],
                            preferred_element_type=jnp.float32)
                    + tt3_ref[...])


def _k5(e2_f, M5, tt3_tiled):
    return pl.pallas_call(
        _k5_body,
        grid=(_GRID,),
        in_specs=[
            pl.BlockSpec((_EB, 128), lambda i: (i, 0)),
            pl.BlockSpec((128, 1024), lambda i: (0, 0)),
            pl.BlockSpec((1, 1024), lambda i: (0, 0)),
        ],
        out_specs=pl.BlockSpec((_EB, 1024), lambda i: (i, 0)),
        out_shape=jax.ShapeDtypeStruct((_EF8, 1024), jnp.float32),
    )(e2_f, M5, tt3_tiled)


def _k7_body(shp_ref, cnt_ref, u_ref, batch_ref, wn2x_ref, wn2u_ref,
             gn2_ref, bn2_ref, an2_ref, wgu_ref, wgx_ref,
             gg_ref, bg_ref, ag_ref,
             x2_ref, u2_ref):
    sh = shp_ref[0] + shp_ref[1]
    hm = sh / jnp.maximum(cnt_ref[...], 1.0)
    oh = (batch_ref[...] == lax.broadcasted_iota(jnp.int32, (1, _NG), 1)
          ).astype(jnp.float32)
    ub = jnp.dot(oh, u_ref[...], preferred_element_type=jnp.float32)
    y = (jnp.dot(hm, wn2x_ref[...].T, preferred_element_type=jnp.float32)
         + jnp.dot(ub, wn2u_ref[...].T, preferred_element_type=jnp.float32))
    m = jnp.mean(y, axis=0, keepdims=True)
    yc = y - m
    v = jnp.mean(yc * yc, axis=0, keepdims=True)
    y = yc * lax.rsqrt(v + 1e-5) * gn2_ref[...] + bn2_ref[...]
    a2 = an2_ref[0, 0]
    x2 = jnp.maximum(y, 0.0) + a2 * jnp.minimum(y, 0.0)
    x2_ref[...] = x2

    segn = lax.dot_general(oh, x2, (((0,), (0,)), ((), ())),
                           preferred_element_type=jnp.float32)
    cntb = jnp.sum(oh, axis=0)[:, None]
    segm = segn / jnp.maximum(cntb, 1.0)
    yg = (jnp.dot(u_ref[...], wgu_ref[...].T, preferred_element_type=jnp.float32)
          + jnp.dot(segm, wgx_ref[...].T, preferred_element_type=jnp.float32))
    mg = jnp.mean(yg, axis=0, keepdims=True)
    ygc = yg - mg
    vg = jnp.mean(ygc * ygc, axis=0, keepdims=True)
    yg = ygc * lax.rsqrt(vg + 1e-5) * gg_ref[...] + bg_ref[...]
    ag = ag_ref[0, 0]
    u2_ref[...] = jnp.maximum(yg, 0.0) + ag * jnp.minimum(yg, 0.0)


def _k7(shP, cnt_row, u, batch2d, Wn2x, Wn2u, gn2, bn2, an2,
        Wgu, Wgx, gg, bg, ag):
    return pl.pallas_call(
        _k7_body,
        out_shape=(
            jax.ShapeDtypeStruct((_N, _NF), jnp.float32),
            jax.ShapeDtypeStruct((_NG, _GF), jnp.float32),
        ),
    )(shP, cnt_row, u, batch2d, Wn2x, Wn2u, gn2, bn2, an2,
      Wgu, Wgx, gg, bg, ag)


def _k6_body(xas_hbm, w_hbm, row_hbm, col_hbm, cvec_hbm, out_hbm,
             ridx_v, cidx_v, g_v, w_v, cv_v, zero_v, acc_sh, sem):
    cid = lax.axis_index("c")
    sid = lax.axis_index("s")
    wid = cid * 16 + sid

    pltpu.sync_copy(cvec_hbm, cv_v)
    _zero_acc(zero_v, acc_sh, sid, _NF)
    plsc.subcore_barrier()
    c1 = cv_v[0, :]
    c2 = cv_v[1, :]

    def chunk(j, _):
        base = wid * _EW + j * _CH
        pltpu.sync_copy(row_hbm.at[pl.ds(base, _CH)], ridx_v)
        pltpu.sync_copy(col_hbm.at[pl.ds(base, _CH)], cidx_v)
        pltpu.async_copy(xas_hbm.at[cidx_v], g_v, sem).wait()
        pltpu.sync_copy(w_hbm.at[pl.ds(base, _CH)], w_v)

        def edge(e, _):
            for l in range(8):
                sl = pl.ds(16 * l, 16)
                z = g_v[e, sl] + w_v[e, sl]
                w_v[e, sl] = c1 * z + c2 * jnp.abs(z)
            return 0
        lax.fori_loop(0, _CH, edge, 0)
        pltpu.sync_copy(w_v, acc_sh.at[ridx_v], add=True)
        return 0
    lax.fori_loop(0, _NCH, chunk, 0)
    plsc.subcore_barrier()

    _read_acc(acc_sh, out_hbm, cid, sid)


_k6 = functools.partial(
    pl.kernel, _k6_body, mesh=_SC_MESH,
    out_type=jax.ShapeDtypeStruct((2, _N, _NF), jnp.float32),
    scratch_types=[
        pltpu.VMEM((_CH,), jnp.int32),
        pltpu.VMEM((_CH,), jnp.int32),
        pltpu.VMEM((_CH, _NF), jnp.float32),
        pltpu.VMEM((_CH, _NF), jnp.float32),
        pltpu.VMEM((2, 16), jnp.float32),
        pltpu.VMEM((48, _NF), jnp.float32),
        pltpu.VMEM_SHARED((_N, _NF), jnp.float32),
        pltpu.SemaphoreType.DMA,
    ],
)()


def kernel(x, edge_index, edge_attr, u, batch,
           We, ge, be, ae,
           Wn1, gn1, bn1, an1,
           Wn2, gn2, bn2, an2,
           Wg, gg, bg, ag):
    row = edge_index[0]
    col = edge_index[1]
    batch2d = batch[:, None].astype(jnp.int32)

    We_x = We[:, :_NF]
    We_e = We[:, _NF:_NF + _EF]
    We_u = We[:, _NF + _EF:]
    A = Wn1[:, :_NF]
    Bt = Wn1[:, _NF:].T
    eye8 = jnp.eye(8, dtype=jnp.float32)
    M0 = jnp.einsum('ij,fg->ifjg', eye8, We_e.T).reshape(128, 128)
    Wn2x = Wn2[:, :_NF]
    Wn2u = Wn2[:, _NF:]
    Wgu = Wg[:, :_GF]
    Wgx = Wg[:, _GF:]
    r2 = lambda v: v[None, :]
    r11 = lambda s: jnp.asarray(s, jnp.float32).reshape(1, 1)

    t1, ta, xa = _k0a(x, u, batch2d, We_x, We_u, A)
    ea_f = _k0b(edge_attr.reshape(_EF8, 128), M0)
    ea = ea_f.reshape(_E, _EF)

    p = ta[row] + t1[col] + ea
    sp = jnp.sum(p, axis=0)
    spp = jnp.sum(p * p, axis=0)
    ones_e = jnp.ones((_E,), jnp.float32)
    cnt_row = jax.ops.segment_sum(ones_e, row, num_segments=_N)
    cnt_col = jax.ops.segment_sum(ones_e, col, num_segments=_N)
    cntP = jnp.zeros((32, 2, _N), jnp.float32)
    cntP = cntP.at[0, 0].set(cnt_row).at[0, 1].set(cnt_col)

    mean1 = sp / _E
    var1 = spp / _E - mean1 * mean1
    s1 = ge * lax.rsqrt(var1 + 1e-5)
    t1bn = be - mean1 * s1

    e2_f, R, se_row = _k2(p.reshape(_EF8, 128),
                          jnp.tile(s1, 8)[None, :],
                          jnp.tile(t1bn, 8)[None, :], r11(ae))
    e2 = e2_f.reshape(_E, _EF)
    S2 = jnp.einsum('ifig->fg', R.reshape(8, 16, 8, 16))
    sum_e2 = jnp.sum(se_row.reshape(8, 16), axis=0, keepdims=True)

    sc2 = jax.ops.segment_sum(e2, col, num_segments=_N)
    sc2P = jnp.zeros((2, _N, _EF), jnp.float32).at[0].set(sc2)

    xas, s3, tt3, cnt2 = _k4(cntP, xa, sc2P, S2, sum_e2, Bt, r2(gn1), r2(bn1))

    Bs_t = Bt * s3.reshape(-1)[None, :]
    M5 = jnp.einsum('ji,of->jfio', eye8,
                    Bs_t.T).reshape(128, 1024)
    w_f = _k5(e2_f, M5, jnp.tile(tt3.reshape(-1), 8)[None, :])
    w = w_f.reshape(_E, _NF)

    c1 = (1.0 + an1) * 0.5
    c2 = (1.0 - an1) * 0.5
    cvec = jnp.stack([jnp.full((16,), c1, jnp.float32),
                      jnp.full((16,), c2, jnp.float32)])
    shP = _k6(xas, w, row.astype(jnp.int32), col.astype(jnp.int32), cvec)

    x2, u2 = _k7(shP, cnt2[0][:, None], u, batch2d, Wn2x, Wn2u,
                 r2(gn2), r2(bn2), r11(an2), Wgu, Wgx,
                 r2(gg), r2(bg), r11(ag))
    return (x2, e2, u2)


# speedup vs baseline: 1.8964x; 1.4801x over previous
import functools

import jax
import jax.numpy as jnp
from jax import lax
from jax.experimental import pallas as pl
from jax.experimental.pallas import tpu as pltpu
from jax.experimental.pallas import tpu_sc as plsc

_N = 10000
_E = 320000
_NG = 8
_NF = 128
_EF = 16
_GF = 16

_EB = 1000
_EF8 = _E // 8
_GRID = _EF8 // _EB

_NW = 32
_EW = _E // _NW
_CH = 80
_NCH = _EW // _CH
_NSL = _N // 16
_SC_MESH = plsc.VectorSubcoreMesh(core_axis_name="c", subcore_axis_name="s")


def _prelu(z, a):
    return jnp.maximum(z, 0.0) + a * jnp.minimum(z, 0.0)


def _k0a_body(x_ref, u_ref, batch_ref, wex_ref, weu_ref, wa_ref,
              tt_ref, xa_ref):
    x = x_ref[...]
    t1 = 0.5 * jnp.dot(x, wex_ref[...].T, preferred_element_type=jnp.float32)
    oh = (batch_ref[...] == lax.broadcasted_iota(jnp.int32, (1, _NG), 1)
          ).astype(jnp.float32)
    ub = jnp.dot(oh, u_ref[...], preferred_element_type=jnp.float32)
    ta = t1 + jnp.dot(ub, weu_ref[...].T, preferred_element_type=jnp.float32)
    xa = jnp.dot(x, wa_ref[...].T, preferred_element_type=jnp.float32)
    tt_ref[...] = jnp.concatenate(
        [ta, t1, jnp.zeros((_N, _NF - 2 * _EF), jnp.float32)], axis=1)
    xa_ref[...] = xa


def _k0a(x, u, batch2d, We_x, We_u, A):
    return pl.pallas_call(
        _k0a_body,
        out_shape=(
            jax.ShapeDtypeStruct((_N, _NF), jnp.float32),
            jax.ShapeDtypeStruct((_N, _NF), jnp.float32),
        ),
    )(x, u, batch2d, We_x, We_u, A)


def _k1_body(tt_hbm, ea_hbm, row_hbm, col_hbm,
             p_hbm, st_hbm,
             ridx_v, cidx_v, gr_v, gc_v, ea_v, p_v, st_v, sem):
    cid = lax.axis_index("c")
    sid = lax.axis_index("s")
    wid = cid * 16 + sid

    st_v[0, :] = jnp.zeros((16,), jnp.float32)
    st_v[1, :] = jnp.zeros((16,), jnp.float32)

    def chunk(j, _):
        base = wid * _EW + j * _CH
        pltpu.sync_copy(row_hbm.at[pl.ds(base, _CH)], ridx_v)
        pltpu.sync_copy(col_hbm.at[pl.ds(base, _CH)], cidx_v)
        pltpu.async_copy(tt_hbm.at[ridx_v], gr_v, sem).wait()
        pltpu.async_copy(tt_hbm.at[cidx_v], gc_v, sem).wait()
        pltpu.sync_copy(ea_hbm.at[pl.ds(base, _CH)], ea_v)

        def edge(e, _):
            p = gr_v[e, pl.ds(0, 16)] + gc_v[e, pl.ds(16, 16)] + ea_v[e, :]
            p_v[e, :] = p
            st_v[0, :] = st_v[0, :] + p
            st_v[1, :] = st_v[1, :] + p * p
            return 0
        lax.fori_loop(0, _CH, edge, 0)
        pltpu.sync_copy(p_v, p_hbm.at[pl.ds(base, _CH)])
        return 0

    lax.fori_loop(0, _NCH, chunk, 0)
    pltpu.sync_copy(st_v, st_hbm.at[wid])


_k1 = functools.partial(
    pl.kernel, _k1_body, mesh=_SC_MESH,
    out_type=(
        jax.ShapeDtypeStruct((_E, _EF), jnp.float32),
        jax.ShapeDtypeStruct((_NW, 2, 16), jnp.float32),
    ),
    scratch_types=[
        pltpu.VMEM((_CH,), jnp.int32),
        pltpu.VMEM((_CH,), jnp.int32),
        pltpu.VMEM((_CH, _NF), jnp.float32),
        pltpu.VMEM((_CH, _NF), jnp.float32),
        pltpu.VMEM((_CH, _EF), jnp.float32),
        pltpu.VMEM((_CH, _EF), jnp.float32),
        pltpu.VMEM((2, 16), jnp.float32),
        pltpu.SemaphoreType.DMA,
    ],
)()


def _k0b_body(eattr_ref, m0_ref, out_ref):
    out_ref[...] = jnp.dot(eattr_ref[...], m0_ref[...],
                           preferred_element_type=jnp.float32)


def _k0b(eattr_f, M0):
    return pl.pallas_call(
        _k0b_body,
        grid=(_GRID,),
        in_specs=[
            pl.BlockSpec((_EB, 128), lambda i: (i, 0)),
            pl.BlockSpec((128, 128), lambda i: (0, 0)),
        ],
        out_specs=pl.BlockSpec((_EB, 128), lambda i: (i, 0)),
        out_shape=jax.ShapeDtypeStruct((_EF8, 128), jnp.float32),
    )(eattr_f, M0)


def _k2_body(p_ref, s1_ref, t1_ref, a_ref, e2_ref, r_ref, se_ref):
    i = pl.program_id(0)
    y = p_ref[...] * s1_ref[...] + t1_ref[...]
    a = a_ref[0, 0]
    e2 = jnp.maximum(y, 0.0) + a * jnp.minimum(y, 0.0)
    e2_ref[...] = e2

    @pl.when(i == 0)
    def _():
        r_ref[...] = jnp.zeros_like(r_ref)
        se_ref[...] = jnp.zeros_like(se_ref)

    r_ref[...] += lax.dot_general(e2, e2, (((0,), (0,)), ((), ())),
                                  preferred_element_type=jnp.float32)
    se_ref[...] += jnp.sum(e2, axis=0, keepdims=True)


def _k2(p_f, s1_tiled, t1_tiled, ae):
    return pl.pallas_call(
        _k2_body,
        grid=(_GRID,),
        in_specs=[
            pl.BlockSpec((_EB, 128), lambda i: (i, 0)),
            pl.BlockSpec((1, 128), lambda i: (0, 0)),
            pl.BlockSpec((1, 128), lambda i: (0, 0)),
            pl.BlockSpec((1, 1), lambda i: (0, 0)),
        ],
        out_specs=(
            pl.BlockSpec((_EB, 128), lambda i: (i, 0)),
            pl.BlockSpec((128, 128), lambda i: (0, 0)),
            pl.BlockSpec((1, 128), lambda i: (0, 0)),
        ),
        out_shape=(
            jax.ShapeDtypeStruct((_EF8, 128), jnp.float32),
            jax.ShapeDtypeStruct((128, 128), jnp.float32),
            jax.ShapeDtypeStruct((1, 128), jnp.float32),
        ),
    )(p_f, s1_tiled, t1_tiled, ae)


def _k4_body(cntp_ref, xa_ref, sc2p_ref, s2_ref, se2_ref, bt_ref,
             gn1_ref, bn1_ref,
             xas_ref, s3_ref, tt3_ref, cnt2_ref):
    cnt2 = jnp.sum(cntp_ref[...], axis=0)
    cnt2_ref[...] = cnt2
    cc = cnt2[1:2, :]
    xa = xa_ref[...]
    sc2 = sc2p_ref[0] + sc2p_ref[1]
    bt = bt_ref[...]
    sum_h = (jnp.dot(cc, xa, preferred_element_type=jnp.float32)
             + jnp.dot(se2_ref[...], bt, preferred_element_type=jnp.float32))
    sq1 = jnp.dot(cc, xa * xa, preferred_element_type=jnp.float32)
    ct = lax.dot_general(sc2, xa, (((0,), (0,)), ((), ())),
                         preferred_element_type=jnp.float32)
    sq2 = 2.0 * jnp.sum(bt * ct, axis=0, keepdims=True)
    q = jnp.dot(s2_ref[...], bt, preferred_element_type=jnp.float32)
    sq3 = jnp.sum(bt * q, axis=0, keepdims=True)
    inv_e = 1.0 / _E
    mean3 = sum_h * inv_e
    var3 = (sq1 + sq2 + sq3) * inv_e - mean3 * mean3
    s3 = gn1_ref[...] * lax.rsqrt(var3 + 1e-5)
    tt3 = bn1_ref[...] - mean3 * s3
    s3_ref[...] = s3
    tt3_ref[...] = tt3
    xas_ref[...] = xa * s3


def _k4(cntP, xa, sc2P, S2, sum_e2, Bt, gn1, bn1):
    return pl.pallas_call(
        _k4_body,
        out_shape=(
            jax.ShapeDtypeStruct((_N, _NF), jnp.float32),
            jax.ShapeDtypeStruct((1, _NF), jnp.float32),
            jax.ShapeDtypeStruct((1, _NF), jnp.float32),
            jax.ShapeDtypeStruct((2, _N), jnp.float32),
        ),
    )(cntP, xa, sc2P, S2, sum_e2, Bt, gn1, bn1)


def _zero_acc(zero_v, acc_sh, sid, width):
    def zrow(i, _):
        zero_v[i, :] = jnp.zeros((width,), jnp.float32)
        return 0
    lax.fori_loop(0, 48, zrow, 0)

    def zchunk(k, _):
        pltpu.sync_copy(zero_v, acc_sh.at[pl.ds(sid * 624 + k * 48, 48)])
        return 0
    lax.fori_loop(0, 13, zchunk, 0)

    @pl.when(sid == 15)
    def _():
        pltpu.sync_copy(zero_v.at[pl.ds(0, 16)], acc_sh.at[pl.ds(9984, 16)])


def _read_acc(acc_sh, out_hbm, cid, sid):
    def rchunk(k, _):
        off = sid * 624 + k * 48
        pltpu.sync_copy(acc_sh.at[pl.ds(off, 48)],
                        out_hbm.at[cid, pl.ds(off, 48)])
        return 0
    lax.fori_loop(0, 13, rchunk, 0)

    @pl.when(sid == 15)
    def _():
        pltpu.sync_copy(acc_sh.at[pl.ds(9984, 16)],
                        out_hbm.at[cid, pl.ds(9984, 16)])


def _k3_body(e2_hbm, col_hbm, out_hbm, idx_v, e2_v, zero_v, acc_sh, sem):
    cid = lax.axis_index("c")
    sid = lax.axis_index("s")
    wid = cid * 16 + sid

    _zero_acc(zero_v, acc_sh, sid, _EF)
    plsc.subcore_barrier()

    def chunk(j, _):
        base = wid * _EW + j * _CH
        pltpu.sync_copy(col_hbm.at[pl.ds(base, _CH)], idx_v)
        pltpu.sync_copy(e2_hbm.at[pl.ds(base, _CH)], e2_v)
        pltpu.sync_copy(e2_v, acc_sh.at[idx_v], add=True)
        return 0
    lax.fori_loop(0, _NCH, chunk, 0)
    plsc.subcore_barrier()

    _read_acc(acc_sh, out_hbm, cid, sid)


_k3 = functools.partial(
    pl.kernel, _k3_body, mesh=_SC_MESH,
    out_type=jax.ShapeDtypeStruct((2, _N, _EF), jnp.float32),
    scratch_types=[
        pltpu.VMEM((_CH,), jnp.int32),
        pltpu.VMEM((_CH, _EF), jnp.float32),
        pltpu.VMEM((48, _EF), jnp.float32),
        pltpu.VMEM_SHARED((_N, _EF), jnp.float32),
        pltpu.SemaphoreType.DMA,
    ],
)()


def _k5_body(e2_ref, m5_ref, tt3_ref, out_ref):
    out_ref[...] = (jnp.dot(e2_ref[...], m5_ref[...],
                            preferred_element_type=jnp.float32)
                    + tt3_ref[...])


def _k5(e2_f, M5, tt3_tiled):
    return pl.pallas_call(
        _k5_body,
        grid=(_GRID,),
        in_specs=[
            pl.BlockSpec((_EB, 128), lambda i: (i, 0)),
            pl.BlockSpec((128, 1024), lambda i: (0, 0)),
            pl.BlockSpec((1, 1024), lambda i: (0, 0)),
        ],
        out_specs=pl.BlockSpec((_EB, 1024), lambda i: (i, 0)),
        out_shape=jax.ShapeDtypeStruct((_EF8, 1024), jnp.float32),
    )(e2_f, M5, tt3_tiled)


def _k7_body(shp_ref, cnt_ref, u_ref, batch_ref, wn2x_ref, wn2u_ref,
             gn2_ref, bn2_ref, an2_ref, wgu_ref, wgx_ref,
             gg_ref, bg_ref, ag_ref,
             x2_ref, u2_ref):
    sh = shp_ref[0] + shp_ref[1]
    hm = sh / jnp.maximum(cnt_ref[...], 1.0)
    oh = (batch_ref[...] == lax.broadcasted_iota(jnp.int32, (1, _NG), 1)
          ).astype(jnp.float32)
    ub = jnp.dot(oh, u_ref[...], preferred_element_type=jnp.float32)
    y = (jnp.dot(hm, wn2x_ref[...].T, preferred_element_type=jnp.float32)
         + jnp.dot(ub, wn2u_ref[...].T, preferred_element_type=jnp.float32))
    m = jnp.mean(y, axis=0, keepdims=True)
    yc = y - m
    v = jnp.mean(yc * yc, axis=0, keepdims=True)
    y = yc * lax.rsqrt(v + 1e-5) * gn2_ref[...] + bn2_ref[...]
    a2 = an2_ref[0, 0]
    x2 = jnp.maximum(y, 0.0) + a2 * jnp.minimum(y, 0.0)
    x2_ref[...] = x2

    segn = lax.dot_general(oh, x2, (((0,), (0,)), ((), ())),
                           preferred_element_type=jnp.float32)
    cntb = jnp.sum(oh, axis=0)[:, None]
    segm = segn / jnp.maximum(cntb, 1.0)
    yg = (jnp.dot(u_ref[...], wgu_ref[...].T, preferred_element_type=jnp.float32)
          + jnp.dot(segm, wgx_ref[...].T, preferred_element_type=jnp.float32))
    mg = jnp.mean(yg, axis=0, keepdims=True)
    ygc = yg - mg
    vg = jnp.mean(ygc * ygc, axis=0, keepdims=True)
    yg = ygc * lax.rsqrt(vg + 1e-5) * gg_ref[...] + bg_ref[...]
    ag = ag_ref[0, 0]
    u2_ref[...] = jnp.maximum(yg, 0.0) + ag * jnp.minimum(yg, 0.0)


def _k7(shP, cnt_row, u, batch2d, Wn2x, Wn2u, gn2, bn2, an2,
        Wgu, Wgx, gg, bg, ag):
    return pl.pallas_call(
        _k7_body,
        out_shape=(
            jax.ShapeDtypeStruct((_N, _NF), jnp.float32),
            jax.ShapeDtypeStruct((_NG, _GF), jnp.float32),
        ),
    )(shP, cnt_row, u, batch2d, Wn2x, Wn2u, gn2, bn2, an2,
      Wgu, Wgx, gg, bg, ag)


def _k6_body(xas_hbm, w_hbm, row_hbm, col_hbm, cvec_hbm, out_hbm,
             ridx_v, cidx_v, g_v, w_v, cv_v, zero_v, acc_sh, sem):
    cid = lax.axis_index("c")
    sid = lax.axis_index("s")
    wid = cid * 16 + sid

    pltpu.sync_copy(cvec_hbm, cv_v)
    _zero_acc(zero_v, acc_sh, sid, _NF)
    plsc.subcore_barrier()
    c1 = cv_v[0, :]
    c2 = cv_v[1, :]

    def chunk(j, _):
        base = wid * _EW + j * _CH
        pltpu.sync_copy(row_hbm.at[pl.ds(base, _CH)], ridx_v)
        pltpu.sync_copy(col_hbm.at[pl.ds(base, _CH)], cidx_v)
        pltpu.async_copy(xas_hbm.at[cidx_v], g_v, sem).wait()
        pltpu.sync_copy(w_hbm.at[pl.ds(base, _CH)], w_v)

        def edge(e, _):
            for l in range(8):
                sl = pl.ds(16 * l, 16)
                z = g_v[e, sl] + w_v[e, sl]
                w_v[e, sl] = c1 * z + c2 * jnp.abs(z)
            return 0
        lax.fori_loop(0, _CH, edge, 0)
        pltpu.sync_copy(w_v, acc_sh.at[ridx_v], add=True)
        return 0
    lax.fori_loop(0, _NCH, chunk, 0)
    plsc.subcore_barrier()

    _read_acc(acc_sh, out_hbm, cid, sid)


_k6 = functools.partial(
    pl.kernel, _k6_body, mesh=_SC_MESH,
    out_type=jax.ShapeDtypeStruct((2, _N, _NF), jnp.float32),
    scratch_types=[
        pltpu.VMEM((_CH,), jnp.int32),
        pltpu.VMEM((_CH,), jnp.int32),
        pltpu.VMEM((_CH, _NF), jnp.float32),
        pltpu.VMEM((_CH, _NF), jnp.float32),
        pltpu.VMEM((2, 16), jnp.float32),
        pltpu.VMEM((48, _NF), jnp.float32),
        pltpu.VMEM_SHARED((_N, _NF), jnp.float32),
        pltpu.SemaphoreType.DMA,
    ],
)()


def kernel(x, edge_index, edge_attr, u, batch,
           We, ge, be, ae,
           Wn1, gn1, bn1, an1,
           Wn2, gn2, bn2, an2,
           Wg, gg, bg, ag):
    row = edge_index[0]
    col = edge_index[1]
    batch2d = batch[:, None].astype(jnp.int32)

    We_x = We[:, :_NF]
    We_e = We[:, _NF:_NF + _EF]
    We_u = We[:, _NF + _EF:]
    A = Wn1[:, :_NF]
    Bt = Wn1[:, _NF:].T
    eye8 = jnp.eye(8, dtype=jnp.float32)
    M0 = jnp.einsum('ij,fg->ifjg', eye8, We_e.T).reshape(128, 128)
    Wn2x = Wn2[:, :_NF]
    Wn2u = Wn2[:, _NF:]
    Wgu = Wg[:, :_GF]
    Wgx = Wg[:, _GF:]
    r2 = lambda v: v[None, :]
    r11 = lambda s: jnp.asarray(s, jnp.float32).reshape(1, 1)

    tt, xa = _k0a(x, u, batch2d, We_x, We_u, A)
    ea_f = _k0b(edge_attr.reshape(_EF8, 128), M0)
    ea = ea_f.reshape(_E, _EF)

    row32 = row.astype(jnp.int32)
    col32 = col.astype(jnp.int32)
    p, stP = _k1(tt, ea, row32, col32)
    sp = jnp.sum(stP[:, 0], axis=0)
    spp = jnp.sum(stP[:, 1], axis=0)
    ones_e = jnp.ones((_E,), jnp.float32)
    cnt_row = jax.ops.segment_sum(ones_e, row, num_segments=_N)
    cnt_col = jax.ops.segment_sum(ones_e, col, num_segments=_N)
    cntP = jnp.zeros((_NW, 2, _N), jnp.float32)
    cntP = cntP.at[0, 0].set(cnt_row).at[0, 1].set(cnt_col)

    mean1 = sp / _E
    var1 = spp / _E - mean1 * mean1
    s1 = ge * lax.rsqrt(var1 + 1e-5)
    t1bn = be - mean1 * s1

    e2_f, R, se_row = _k2(p.reshape(_EF8, 128),
                          jnp.tile(s1, 8)[None, :],
                          jnp.tile(t1bn, 8)[None, :], r11(ae))
    e2 = e2_f.reshape(_E, _EF)
    S2 = jnp.einsum('ifig->fg', R.reshape(8, 16, 8, 16))
    sum_e2 = jnp.sum(se_row.reshape(8, 16), axis=0, keepdims=True)

    sc2 = jax.ops.segment_sum(e2, col, num_segments=_N)
    sc2P = jnp.zeros((2, _N, _EF), jnp.float32).at[0].set(sc2)

    xas, s3, tt3, cnt2 = _k4(cntP, xa, sc2P, S2, sum_e2, Bt, r2(gn1), r2(bn1))

    Bs_t = Bt * s3.reshape(-1)[None, :]
    M5 = jnp.einsum('ji,of->jfio', eye8,
                    Bs_t.T).reshape(128, 1024)
    w_f = _k5(e2_f, M5, jnp.tile(tt3.reshape(-1), 8)[None, :])
    w = w_f.reshape(_E, _NF)

    c1 = (1.0 + an1) * 0.5
    c2 = (1.0 - an1) * 0.5
    cvec = jnp.stack([jnp.full((16,), c1, jnp.float32),
                      jnp.full((16,), c2, jnp.float32)])
    shP = _k6(xas, w, row32, col32, cvec)

    x2, u2 = _k7(shP, cnt2[0][:, None], u, batch2d, Wn2x, Wn2u,
                 r2(gn2), r2(bn2), r11(an2), Wgu, Wgx,
                 r2(gg), r2(bg), r11(ag))
    return (x2, e2, u2)


# speedup vs baseline: 1.8967x; 1.0001x over previous
import functools

import jax
import jax.numpy as jnp
from jax import lax
from jax.experimental import pallas as pl
from jax.experimental.pallas import tpu as pltpu
from jax.experimental.pallas import tpu_sc as plsc

_N = 10000
_E = 320000
_NG = 8
_NF = 128
_EF = 16
_GF = 16

_EB = 1000
_EF8 = _E // 8
_GRID = _EF8 // _EB

_NW = 32
_EW = _E // _NW
_CH = 80
_NCH = _EW // _CH
_SC_MESH = plsc.VectorSubcoreMesh(core_axis_name="c", subcore_axis_name="s")


def _k0a_body(x_ref, u_ref, batch_ref, wex_ref, weu_ref, wa_ref,
              tt_ref, xa_ref):
    x = x_ref[...]
    t1 = 0.5 * jnp.dot(x, wex_ref[...].T, preferred_element_type=jnp.float32)
    oh = (batch_ref[...] == lax.broadcasted_iota(jnp.int32, (1, _NG), 1)
          ).astype(jnp.float32)
    ub = jnp.dot(oh, u_ref[...], preferred_element_type=jnp.float32)
    ta = t1 + jnp.dot(ub, weu_ref[...].T, preferred_element_type=jnp.float32)
    xa = jnp.dot(x, wa_ref[...].T, preferred_element_type=jnp.float32)
    tt_ref[...] = jnp.concatenate(
        [ta, t1, jnp.zeros((_N, _NF - 2 * _EF), jnp.float32)], axis=1)
    xa_ref[...] = xa


def _k0a(x, u, batch2d, We_x, We_u, A):
    return pl.pallas_call(
        _k0a_body,
        out_shape=(
            jax.ShapeDtypeStruct((_N, _NF), jnp.float32),
            jax.ShapeDtypeStruct((_N, _NF), jnp.float32),
        ),
    )(x, u, batch2d, We_x, We_u, A)


def _k1_body(tt_hbm, ea_hbm, row_hbm, col_hbm,
             p_hbm, st_hbm,
             ridx_v, cidx_v, gr_v, gc_v, ea_v, p_v, st_v, sem):
    cid = lax.axis_index("c")
    sid = lax.axis_index("s")
    wid = cid * 16 + sid

    st_v[0, :] = jnp.zeros((16,), jnp.float32)
    st_v[1, :] = jnp.zeros((16,), jnp.float32)

    def chunk(j, _):
        base = wid * _EW + j * _CH
        pltpu.sync_copy(row_hbm.at[pl.ds(base, _CH)], ridx_v)
        pltpu.sync_copy(col_hbm.at[pl.ds(base, _CH)], cidx_v)
        pltpu.async_copy(tt_hbm.at[ridx_v], gr_v, sem).wait()
        pltpu.async_copy(tt_hbm.at[cidx_v], gc_v, sem).wait()
        pltpu.sync_copy(ea_hbm.at[pl.ds(base, _CH)], ea_v)

        def edge(e, _):
            p = gr_v[e, pl.ds(0, 16)] + gc_v[e, pl.ds(16, 16)] + ea_v[e, :]
            p_v[e, :] = p
            st_v[0, :] = st_v[0, :] + p
            st_v[1, :] = st_v[1, :] + p * p
            return 0
        lax.fori_loop(0, _CH, edge, 0)
        pltpu.sync_copy(p_v, p_hbm.at[pl.ds(base, _CH)])
        return 0

    lax.fori_loop(0, _NCH, chunk, 0)
    pltpu.sync_copy(st_v, st_hbm.at[wid])


_k1 = functools.partial(
    pl.kernel, _k1_body, mesh=_SC_MESH,
    out_type=(
        jax.ShapeDtypeStruct((_E, _EF), jnp.float32),
        jax.ShapeDtypeStruct((_NW, 2, 16), jnp.float32),
    ),
    scratch_types=[
        pltpu.VMEM((_CH,), jnp.int32),
        pltpu.VMEM((_CH,), jnp.int32),
        pltpu.VMEM((_CH, _NF), jnp.float32),
        pltpu.VMEM((_CH, _NF), jnp.float32),
        pltpu.VMEM((_CH, _EF), jnp.float32),
        pltpu.VMEM((_CH, _EF), jnp.float32),
        pltpu.VMEM((2, 16), jnp.float32),
        pltpu.SemaphoreType.DMA,
    ],
)()


def _k0b_body(eattr_ref, m0_ref, out_ref):
    out_ref[...] = jnp.dot(eattr_ref[...], m0_ref[...],
                           preferred_element_type=jnp.float32)


def _k0b(eattr_f, M0):
    return pl.pallas_call(
        _k0b_body,
        grid=(_GRID,),
        in_specs=[
            pl.BlockSpec((_EB, 128), lambda i: (i, 0)),
            pl.BlockSpec((128, 128), lambda i: (0, 0)),
        ],
        out_specs=pl.BlockSpec((_EB, 128), lambda i: (i, 0)),
        out_shape=jax.ShapeDtypeStruct((_EF8, 128), jnp.float32),
    )(eattr_f, M0)


def _k2_body(p_ref, s1_ref, t1_ref, a_ref, e2_ref, r_ref, se_ref):
    i = pl.program_id(0)
    y = p_ref[...] * s1_ref[...] + t1_ref[...]
    a = a_ref[0, 0]
    e2 = jnp.maximum(y, 0.0) + a * jnp.minimum(y, 0.0)
    e2_ref[...] = e2

    @pl.when(i == 0)
    def _():
        r_ref[...] = jnp.zeros_like(r_ref)
        se_ref[...] = jnp.zeros_like(se_ref)

    r_ref[...] += lax.dot_general(e2, e2, (((0,), (0,)), ((), ())),
                                  preferred_element_type=jnp.float32)
    se_ref[...] += jnp.sum(e2, axis=0, keepdims=True)


def _k2(p_f, s1_tiled, t1_tiled, ae):
    return pl.pallas_call(
        _k2_body,
        grid=(_GRID,),
        in_specs=[
            pl.BlockSpec((_EB, 128), lambda i: (i, 0)),
            pl.BlockSpec((1, 128), lambda i: (0, 0)),
            pl.BlockSpec((1, 128), lambda i: (0, 0)),
            pl.BlockSpec((1, 1), lambda i: (0, 0)),
        ],
        out_specs=(
            pl.BlockSpec((_EB, 128), lambda i: (i, 0)),
            pl.BlockSpec((128, 128), lambda i: (0, 0)),
            pl.BlockSpec((1, 128), lambda i: (0, 0)),
        ),
        out_shape=(
            jax.ShapeDtypeStruct((_EF8, 128), jnp.float32),
            jax.ShapeDtypeStruct((128, 128), jnp.float32),
            jax.ShapeDtypeStruct((1, 128), jnp.float32),
        ),
    )(p_f, s1_tiled, t1_tiled, ae)


def _k4_body(cntp_ref, xa_ref, sc2p_ref, s2_ref, se2_ref, bt_ref,
             gn1_ref, bn1_ref,
             xas_ref, s3_ref, tt3_ref, cnt2_ref):
    cnt2 = jnp.sum(cntp_ref[...], axis=0)
    cnt2_ref[...] = cnt2
    cc = cnt2[1:2, :]
    xa = xa_ref[...]
    sc2 = sc2p_ref[0] + sc2p_ref[1]
    bt = bt_ref[...]
    sum_h = (jnp.dot(cc, xa, preferred_element_type=jnp.float32)
             + jnp.dot(se2_ref[...], bt, preferred_element_type=jnp.float32))
    sq1 = jnp.dot(cc, xa * xa, preferred_element_type=jnp.float32)
    ct = lax.dot_general(sc2, xa, (((0,), (0,)), ((), ())),
                         preferred_element_type=jnp.float32)
    sq2 = 2.0 * jnp.sum(bt * ct, axis=0, keepdims=True)
    q = jnp.dot(s2_ref[...], bt, preferred_element_type=jnp.float32)
    sq3 = jnp.sum(bt * q, axis=0, keepdims=True)
    inv_e = 1.0 / _E
    mean3 = sum_h * inv_e
    var3 = (sq1 + sq2 + sq3) * inv_e - mean3 * mean3
    s3 = gn1_ref[...] * lax.rsqrt(var3 + 1e-5)
    tt3 = bn1_ref[...] - mean3 * s3
    s3_ref[...] = s3
    tt3_ref[...] = tt3
    xas_ref[...] = xa * s3


def _k4(cntP, xa, sc2P, S2, sum_e2, Bt, gn1, bn1):
    return pl.pallas_call(
        _k4_body,
        out_shape=(
            jax.ShapeDtypeStruct((_N, _NF), jnp.float32),
            jax.ShapeDtypeStruct((1, _NF), jnp.float32),
            jax.ShapeDtypeStruct((1, _NF), jnp.float32),
            jax.ShapeDtypeStruct((2, _N), jnp.float32),
        ),
    )(cntP, xa, sc2P, S2, sum_e2, Bt, gn1, bn1)


def _k5_body(e2_ref, m5_ref, tt3_ref, out_ref):
    out_ref[...] = (jnp.dot(e2_ref[...], m5_ref[...],
                            preferred_element_type=jnp.float32)
                    + tt3_ref[...])


def _k5(e2_f, M5, tt3_tiled):
    return pl.pallas_call(
        _k5_body,
        grid=(_GRID,),
        in_specs=[
            pl.BlockSpec((_EB, 128), lambda i: (i, 0)),
            pl.BlockSpec((128, 1024), lambda i: (0, 0)),
            pl.BlockSpec((1, 1024), lambda i: (0, 0)),
        ],
        out_specs=pl.BlockSpec((_EB, 1024), lambda i: (i, 0)),
        out_shape=jax.ShapeDtypeStruct((_EF8, 1024), jnp.float32),
    )(e2_f, M5, tt3_tiled)


def _k7_body(shp_ref, cnt_ref, u_ref, batch_ref, wn2x_ref, wn2u_ref,
             gn2_ref, bn2_ref, an2_ref, wgu_ref, wgx_ref,
             gg_ref, bg_ref, ag_ref,
             x2_ref, u2_ref):
    sh = shp_ref[0] + shp_ref[1]
    hm = sh / jnp.maximum(cnt_ref[...], 1.0)
    oh = (batch_ref[...] == lax.broadcasted_iota(jnp.int32, (1, _NG), 1)
          ).astype(jnp.float32)
    ub = jnp.dot(oh, u_ref[...], preferred_element_type=jnp.float32)
    y = (jnp.dot(hm, wn2x_ref[...].T, preferred_element_type=jnp.float32)
         + jnp.dot(ub, wn2u_ref[...].T, preferred_element_type=jnp.float32))
    m = jnp.mean(y, axis=0, keepdims=True)
    yc = y - m
    v = jnp.mean(yc * yc, axis=0, keepdims=True)
    y = yc * lax.rsqrt(v + 1e-5) * gn2_ref[...] + bn2_ref[...]
    a2 = an2_ref[0, 0]
    x2 = jnp.maximum(y, 0.0) + a2 * jnp.minimum(y, 0.0)
    x2_ref[...] = x2

    segn = lax.dot_general(oh, x2, (((0,), (0,)), ((), ())),
                           preferred_element_type=jnp.float32)
    cntb = jnp.sum(oh, axis=0)[:, None]
    segm = segn / jnp.maximum(cntb, 1.0)
    yg = (jnp.dot(u_ref[...], wgu_ref[...].T, preferred_element_type=jnp.float32)
          + jnp.dot(segm, wgx_ref[...].T, preferred_element_type=jnp.float32))
    mg = jnp.mean(yg, axis=0, keepdims=True)
    ygc = yg - mg
    vg = jnp.mean(ygc * ygc, axis=0, keepdims=True)
    yg = ygc * lax.rsqrt(vg + 1e-5) * gg_ref[...] + bg_ref[...]
    ag = ag_ref[0, 0]
    u2_ref[...] = jnp.maximum(yg, 0.0) + ag * jnp.minimum(yg, 0.0)


def _k7(shP, cnt_row, u, batch2d, Wn2x, Wn2u, gn2, bn2, an2,
        Wgu, Wgx, gg, bg, ag):
    return pl.pallas_call(
        _k7_body,
        out_shape=(
            jax.ShapeDtypeStruct((_N, _NF), jnp.float32),
            jax.ShapeDtypeStruct((_NG, _GF), jnp.float32),
        ),
    )(shP, cnt_row, u, batch2d, Wn2x, Wn2u, gn2, bn2, an2,
      Wgu, Wgx, gg, bg, ag)


def _zero_acc(zero_v, acc_sh, sid, width):
    def zrow(i, _):
        zero_v[i, :] = jnp.zeros((width,), jnp.float32)
        return 0
    lax.fori_loop(0, 48, zrow, 0)

    def zchunk(k, _):
        pltpu.sync_copy(zero_v, acc_sh.at[pl.ds(sid * 624 + k * 48, 48)])
        return 0
    lax.fori_loop(0, 13, zchunk, 0)

    @pl.when(sid == 15)
    def _():
        pltpu.sync_copy(zero_v.at[pl.ds(0, 16)], acc_sh.at[pl.ds(9984, 16)])


def _read_acc(acc_sh, out_hbm, cid, sid):
    def rchunk(k, _):
        off = sid * 624 + k * 48
        pltpu.sync_copy(acc_sh.at[pl.ds(off, 48)],
                        out_hbm.at[cid, pl.ds(off, 48)])
        return 0
    lax.fori_loop(0, 13, rchunk, 0)

    @pl.when(sid == 15)
    def _():
        pltpu.sync_copy(acc_sh.at[pl.ds(9984, 16)],
                        out_hbm.at[cid, pl.ds(9984, 16)])


def _k6_body(xas_hbm, w_hbm, row_hbm, col_hbm, cvec_hbm, out_hbm,
             ridx_v, cidx_v, g_v, w_v, cv_v, zero_v, acc_sh, sem):
    cid = lax.axis_index("c")
    sid = lax.axis_index("s")
    wid = cid * 16 + sid

    pltpu.sync_copy(cvec_hbm, cv_v)
    _zero_acc(zero_v, acc_sh, sid, _NF)
    plsc.subcore_barrier()
    c1 = cv_v[0, :]
    c2 = cv_v[1, :]

    def chunk(j, _):
        base = wid * _EW + j * _CH
        pltpu.sync_copy(row_hbm.at[pl.ds(base, _CH)], ridx_v)
        pltpu.sync_copy(col_hbm.at[pl.ds(base, _CH)], cidx_v)
        pltpu.async_copy(xas_hbm.at[cidx_v], g_v, sem).wait()
        pltpu.sync_copy(w_hbm.at[pl.ds(base, _CH)], w_v)

        def edge(e, _):
            for l in range(8):
                sl = pl.ds(16 * l, 16)
                z = g_v[e, sl] + w_v[e, sl]
                w_v[e, sl] = c1 * z + c2 * jnp.abs(z)
            return 0
        lax.fori_loop(0, _CH, edge, 0)
        pltpu.sync_copy(w_v, acc_sh.at[ridx_v], add=True)
        return 0
    lax.fori_loop(0, _NCH, chunk, 0)
    plsc.subcore_barrier()

    _read_acc(acc_sh, out_hbm, cid, sid)


_k6 = functools.partial(
    pl.kernel, _k6_body, mesh=_SC_MESH,
    out_type=jax.ShapeDtypeStruct((2, _N, _NF), jnp.float32),
    scratch_types=[
        pltpu.VMEM((_CH,), jnp.int32),
        pltpu.VMEM((_CH,), jnp.int32),
        pltpu.VMEM((_CH, _NF), jnp.float32),
        pltpu.VMEM((_CH, _NF), jnp.float32),
        pltpu.VMEM((2, 16), jnp.float32),
        pltpu.VMEM((48, _NF), jnp.float32),
        pltpu.VMEM_SHARED((_N, _NF), jnp.float32),
        pltpu.SemaphoreType.DMA,
    ],
)()


def kernel(x, edge_index, edge_attr, u, batch,
           We, ge, be, ae,
           Wn1, gn1, bn1, an1,
           Wn2, gn2, bn2, an2,
           Wg, gg, bg, ag):
    row = edge_index[0]
    col = edge_index[1]
    batch2d = batch[:, None].astype(jnp.int32)

    We_x = We[:, :_NF]
    We_e = We[:, _NF:_NF + _EF]
    We_u = We[:, _NF + _EF:]
    A = Wn1[:, :_NF]
    Bt = Wn1[:, _NF:].T
    eye8 = jnp.eye(8, dtype=jnp.float32)
    M0 = jnp.einsum('ij,fg->ifjg', eye8, We_e.T).reshape(128, 128)
    Wn2x = Wn2[:, :_NF]
    Wn2u = Wn2[:, _NF:]
    Wgu = Wg[:, :_GF]
    Wgx = Wg[:, _GF:]
    r2 = lambda v: v[None, :]
    r11 = lambda s: jnp.asarray(s, jnp.float32).reshape(1, 1)

    tt, xa = _k0a(x, u, batch2d, We_x, We_u, A)
    ea_f = _k0b(edge_attr.reshape(_EF8, 128), M0)
    ea = ea_f.reshape(_E, _EF)

    row32 = row.astype(jnp.int32)
    col32 = col.astype(jnp.int32)
    p, stP = _k1(tt, ea, row32, col32)
    sp = jnp.sum(stP[:, 0], axis=0)
    spp = jnp.sum(stP[:, 1], axis=0)
    ones_e = jnp.ones((_E,), jnp.float32)
    cnt_row = jax.ops.segment_sum(ones_e, row, num_segments=_N)
    cnt_col = jax.ops.segment_sum(ones_e, col, num_segments=_N)
    cntP = jnp.zeros((_NW, 2, _N), jnp.float32)
    cntP = cntP.at[0, 0].set(cnt_row).at[0, 1].set(cnt_col)

    mean1 = sp / _E
    var1 = spp / _E - mean1 * mean1
    s1 = ge * lax.rsqrt(var1 + 1e-5)
    t1bn = be - mean1 * s1

    e2_f, R, se_row = _k2(p.reshape(_EF8, 128),
                          jnp.tile(s1, 8)[None, :],
                          jnp.tile(t1bn, 8)[None, :], r11(ae))
    e2 = e2_f.reshape(_E, _EF)
    S2 = jnp.einsum('ifig->fg', R.reshape(8, 16, 8, 16))
    sum_e2 = jnp.sum(se_row.reshape(8, 16), axis=0, keepdims=True)

    sc2 = jax.ops.segment_sum(e2, col, num_segments=_N)
    sc2P = jnp.zeros((2, _N, _EF), jnp.float32).at[0].set(sc2)

    xas, s3, tt3, cnt2 = _k4(cntP, xa, sc2P, S2, sum_e2, Bt, r2(gn1), r2(bn1))

    Bs_t = Bt * s3.reshape(-1)[None, :]
    M5 = jnp.einsum('ji,of->jfio', eye8,
                    Bs_t.T).reshape(128, 1024)
    w_f = _k5(e2_f, M5, jnp.tile(tt3.reshape(-1), 8)[None, :])
    w = w_f.reshape(_E, _NF)

    c1 = (1.0 + an1) * 0.5
    c2 = (1.0 - an1) * 0.5
    cvec = jnp.stack([jnp.full((16,), c1, jnp.float32),
                      jnp.full((16,), c2, jnp.float32)])
    shP = _k6(xas, w, row32, col32, cvec)

    x2, u2 = _k7(shP, cnt2[0][:, None], u, batch2d, Wn2x, Wn2u,
                 r2(gn2), r2(bn2), r11(an2), Wgu, Wgx,
                 r2(gg), r2(bg), r11(ag))
    return (x2, e2, u2)
